# Initial kernel scaffold; baseline (speedup 1.0000x reference)
import jax
import jax.numpy as jnp
from jax.experimental import pallas as pl


def kernel(input_ids, text_embedding, positional_embedding):
    raise NotImplementedError("write your pallas kernel here")


# speedup vs baseline: 9.8110x; 9.8110x over previous
import functools

import jax
import jax.numpy as jnp
from jax import lax
from jax.experimental import pallas as pl
from jax.experimental.pallas import tpu as pltpu
from jax.experimental.pallas import tpu_sc as plsc

B = 16384
S = 77
D = 64
EOS = 49407
NC = 2
NS = 16
NW = NC * NS
ROWS_PER_W = B // NW
CB = 8
NCHUNK = ROWS_PER_W // CB
LANES = 16


def _emb_body(ids_hbm, table_hbm, pos_hbm, out_hbm, pos_v, idx_v, rows_v, sem):
    wid = lax.axis_index("s") * NC + lax.axis_index("c")
    base_row = wid * ROWS_PER_W

    pltpu.sync_copy(pos_hbm, pos_v)

    def chunk_body(g, carry):
        row0 = base_row + g * CB
        pltpu.sync_copy(ids_hbm.at[pl.ds(row0, CB)], idx_v)
        for c in range(CB):
            pltpu.async_copy(table_hbm.at[idx_v.at[c]], rows_v.at[c], sem)
        for c in range(CB):
            pltpu.make_async_copy(table_hbm.at[idx_v.at[c]], rows_v.at[c], sem).wait()

        def s_body(s, carry2):
            for j in range(D // LANES):
                p = pos_v[s, pl.ds(j * LANES, LANES)]
                for c in range(CB):
                    rows_v[c, s, pl.ds(j * LANES, LANES)] += p
            return carry2

        lax.fori_loop(0, S, s_body, 0)
        pltpu.sync_copy(rows_v, out_hbm.at[pl.ds(row0, CB)])
        return carry

    lax.fori_loop(0, NCHUNK, chunk_body, 0)


_emb = functools.partial(
    pl.kernel,
    out_type=jax.ShapeDtypeStruct((B, S, D), jnp.float32),
    mesh=plsc.VectorSubcoreMesh(core_axis_name="c", subcore_axis_name="s"),
    scratch_types=[
        pltpu.VMEM((S, D), jnp.float32),
        pltpu.VMEM((CB, S), jnp.int32),
        pltpu.VMEM((CB, S, D), jnp.float32),
        pltpu.SemaphoreType.DMA,
    ],
    compiler_params=pltpu.CompilerParams(use_tc_tiling_on_sc=False),
)(_emb_body)


def _mask_body(ids_ref, m_ref):
    m_ref[...] = ids_ref[...] == EOS


_mask = pl.pallas_call(
    _mask_body,
    out_shape=jax.ShapeDtypeStruct((B, S), jnp.bool_),
)


def kernel(input_ids, text_embedding, positional_embedding):
    ids = input_ids.astype(jnp.int32)
    tokens = _emb(ids, text_embedding, positional_embedding)
    mask = _mask(ids)
    return tokens, mask


# speedup vs baseline: 11.2671x; 1.1484x over previous
import functools

import jax
import jax.numpy as jnp
from jax import lax
from jax.experimental import pallas as pl
from jax.experimental.pallas import tpu as pltpu
from jax.experimental.pallas import tpu_sc as plsc

B = 16384
S = 77
D = 64
EOS = 49407
NC = 2
NS = 16
NW = NC * NS
ROWS_PER_W = B // NW
CB = 4
NCHUNK = ROWS_PER_W // CB
NBUF = 4
LANES = 16


def _emb_body(ids_hbm, table_hbm, pos_hbm, out_hbm, pos_v,
              idx0, idx1, idx2, idx3, rows0, rows1, rows2, rows3,
              sg0, sg1, sg2, sg3, ss0, ss1, ss2, ss3):
    idx = [idx0, idx1, idx2, idx3]
    rows = [rows0, rows1, rows2, rows3]
    sg = [sg0, sg1, sg2, sg3]
    ss = [ss0, ss1, ss2, ss3]

    wid = lax.axis_index("s") * NC + lax.axis_index("c")
    base_row = wid * ROWS_PER_W

    pltpu.sync_copy(pos_hbm, pos_v)

    def start_chunk(b, g):
        row0 = base_row + g * CB
        pltpu.sync_copy(ids_hbm.at[pl.ds(row0, CB)], idx[b])
        for c in range(CB):
            pltpu.async_copy(table_hbm.at[idx[b].at[c]], rows[b].at[c], sg[b])

    def wait_gathers(b):
        for c in range(CB):
            pltpu.make_async_copy(table_hbm.at[idx[b].at[c]], rows[b].at[c],
                                  sg[b]).wait()

    def add_pos(b):
        r = rows[b]

        def s_body(s, carry):
            for j in range(D // LANES):
                p = pos_v[s, pl.ds(j * LANES, LANES)]
                for c in range(CB):
                    r[c, s, pl.ds(j * LANES, LANES)] += p
            return carry

        lax.fori_loop(0, S, s_body, 0)

    def start_store(b, g):
        row0 = base_row + g * CB
        pltpu.async_copy(rows[b], out_hbm.at[pl.ds(row0, CB)], ss[b])

    def wait_store(b, g):
        row0 = base_row + g * CB
        pltpu.make_async_copy(rows[b], out_hbm.at[pl.ds(row0, CB)], ss[b]).wait()

    start_chunk(0, 0)
    start_chunk(1, 1)

    def outer_body(i, carry):
        for bb in range(NBUF):
            g = i * NBUF + bb
            b = bb
            wait_gathers(b)
            add_pos(b)
            start_store(b, g)
            bn = (bb + 2) % NBUF

            @pl.when(g + 2 < NCHUNK)
            def _():
                @pl.when(g >= 2)
                def _():
                    wait_store(bn, g - 2)

                start_chunk(bn, g + 2)

        return carry

    lax.fori_loop(0, NCHUNK // NBUF, outer_body, 0)

    for k in range(NBUF):
        g = NCHUNK - NBUF + k
        wait_store(g % NBUF, g)


_scr_idx = [pltpu.VMEM((CB, S), jnp.int32) for _ in range(NBUF)]
_scr_rows = [pltpu.VMEM((CB, S, D), jnp.float32) for _ in range(NBUF)]
_scr_sem = [pltpu.SemaphoreType.DMA for _ in range(2 * NBUF)]

_emb = functools.partial(
    pl.kernel,
    out_type=jax.ShapeDtypeStruct((B, S, D), jnp.float32),
    mesh=plsc.VectorSubcoreMesh(core_axis_name="c", subcore_axis_name="s"),
    scratch_types=[pltpu.VMEM((S, D), jnp.float32)]
    + _scr_idx + _scr_rows + _scr_sem,
    compiler_params=pltpu.CompilerParams(use_tc_tiling_on_sc=False),
)(_emb_body)


def _mask_body(ids_ref, m_ref):
    m_ref[...] = ids_ref[...] == EOS


_mask = pl.pallas_call(
    _mask_body,
    out_shape=jax.ShapeDtypeStruct((B, S), jnp.bool_),
)


def kernel(input_ids, text_embedding, positional_embedding):
    ids = input_ids.astype(jnp.int32)
    tokens = _emb(ids, text_embedding, positional_embedding)
    mask = _mask(ids)
    return tokens, mask
